# baseline (speedup 1.0000x reference)
import functools
import math

import jax
import jax.numpy as jnp
import numpy as np
from jax import lax
from jax.experimental import pallas as pl
from jax.experimental.pallas import tpu as pltpu
from jax.experimental.pallas import tpu_sc as plsc

_GRID = 4
_STRIDE = 2

_B, _C, _H, _W = 8, 96, 384, 384
_M = 192
_NW = 32
_K = 64
_ROWS_PER_W = (_B * _C * _M) // _NW
_CHUNKS = _ROWS_PER_W // _K
_LANES = 16


def _sampled_axis_idx(key, size, grid, m):
    nblocks = math.ceil(size / grid)
    keys = jax.random.split(key, nblocks)
    perms = jax.vmap(lambda k: jax.random.permutation(k, grid)[:m])(keys)
    idx = (perms + jnp.arange(nblocks)[:, None] * grid).reshape(-1)
    return jnp.clip(jnp.sort(idx), 0, size - 1)


def _index_arrays():
    m = _GRID // _STRIDE
    kr, kc = jax.random.split(jax.random.key(42))
    r = _sampled_axis_idx(kr, _H, _GRID, m).astype(jnp.int32)
    c = _sampled_axis_idx(kc, _W, _GRID, m).astype(jnp.int32)
    planes = jnp.arange(_B * _C, dtype=jnp.int32)[:, None] * _H
    rowidx = (planes + r[None, :]).reshape(-1)
    return rowidx, c


@functools.lru_cache(maxsize=None)
def _static_indices_np():
    with jax.ensure_compile_time_eval():
        rowidx, c = _index_arrays()
        return np.asarray(rowidx), np.asarray(c)


def _static_indices():
    try:
        rowidx_np, c_np = _static_indices_np()
        return jnp.asarray(rowidx_np), jnp.asarray(c_np)
    except Exception:
        return _index_arrays()


def _body(x2, rowidx, cidx, out, idx0, idx1, in0, in1, out0, out1, cv,
          gsem0, gsem1, osem0, osem1):
    wid = lax.axis_index("s") * 2 + lax.axis_index("c")
    base = wid * _ROWS_PER_W

    pltpu.sync_copy(cidx, cv)
    cvecs = [cv[pl.ds(_LANES * j, _LANES)] for j in range(_M // _LANES)]

    def col_gather(src, dst):
        def row_body(r, carry):
            rv = jnp.full((_LANES,), r, jnp.int32)
            vals = [plsc.load_gather(src, [rv, cvecs[j]])
                    for j in range(_M // _LANES)]
            for j, v in enumerate(vals):
                dst[r, pl.ds(_LANES * j, _LANES)] = v
            return carry
        lax.fori_loop(0, _K, row_body, 0)

    def pair_body(g, carry):
        o0 = base + (2 * g) * _K
        o1 = base + (2 * g + 1) * _K
        pltpu.sync_copy(rowidx.at[pl.ds(o0, _K)], idx0)
        d0 = pltpu.async_copy(x2.at[idx0], in0, gsem0)
        pltpu.sync_copy(rowidx.at[pl.ds(o1, _K)], idx1)
        d1 = pltpu.async_copy(x2.at[idx1], in1, gsem1)
        d0.wait()
        col_gather(in0, out0)
        w0 = pltpu.async_copy(out0, out.at[pl.ds(o0, _K)], osem0)
        d1.wait()
        col_gather(in1, out1)
        w1 = pltpu.async_copy(out1, out.at[pl.ds(o1, _K)], osem1)
        w0.wait()
        w1.wait()
        return carry

    lax.fori_loop(0, _CHUNKS // 2, pair_body, 0)


def kernel(x):
    rowidx, cidx = _static_indices()
    x2 = x.reshape(_B * _C * _H, _W)

    run = pl.kernel(
        _body,
        out_type=jax.ShapeDtypeStruct((_B * _C * _M, _M), jnp.float32),
        mesh=plsc.VectorSubcoreMesh(core_axis_name="c", subcore_axis_name="s"),
        compiler_params=pltpu.CompilerParams(
            use_tc_tiling_on_sc=True, needs_layout_passes=False),
        scratch_types=[
            pltpu.VMEM((_K,), jnp.int32),
            pltpu.VMEM((_K,), jnp.int32),
            pltpu.VMEM((_K, _W), jnp.float32),
            pltpu.VMEM((_K, _W), jnp.float32),
            pltpu.VMEM((_K, _M), jnp.float32),
            pltpu.VMEM((_K, _M), jnp.float32),
            pltpu.VMEM((_M,), jnp.int32),
            pltpu.SemaphoreType.DMA,
            pltpu.SemaphoreType.DMA,
            pltpu.SemaphoreType.DMA,
            pltpu.SemaphoreType.DMA,
        ],
    )
    out2 = run(x2, rowidx, cidx)
    return out2.reshape(_B, _C, _M, _M)


# speedup vs baseline: 1.0560x; 1.0560x over previous
import functools
import math

import jax
import jax.numpy as jnp
import numpy as np
from jax import lax
from jax.experimental import pallas as pl
from jax.experimental.pallas import tpu as pltpu
from jax.experimental.pallas import tpu_sc as plsc

_GRID = 4
_STRIDE = 2

_B, _C, _H, _W = 8, 96, 384, 384
_M = 192
_NW = 32
_K = 96
_ROWS_PER_W = (_B * _C * _M) // _NW
_CHUNKS = _ROWS_PER_W // _K
_LANES = 16


def _sampled_axis_idx(key, size, grid, m):
    nblocks = math.ceil(size / grid)
    keys = jax.random.split(key, nblocks)
    perms = jax.vmap(lambda k: jax.random.permutation(k, grid)[:m])(keys)
    idx = (perms + jnp.arange(nblocks)[:, None] * grid).reshape(-1)
    return jnp.clip(jnp.sort(idx), 0, size - 1)


def _index_arrays():
    m = _GRID // _STRIDE
    kr, kc = jax.random.split(jax.random.key(42))
    r = _sampled_axis_idx(kr, _H, _GRID, m).astype(jnp.int32)
    c = _sampled_axis_idx(kc, _W, _GRID, m).astype(jnp.int32)
    planes = jnp.arange(_B * _C, dtype=jnp.int32)[:, None] * _H
    rowidx = (planes + r[None, :]).reshape(-1)
    return rowidx, c


@functools.lru_cache(maxsize=None)
def _static_indices_np():
    with jax.ensure_compile_time_eval():
        rowidx, c = _index_arrays()
        return np.asarray(rowidx), np.asarray(c)


def _static_indices():
    try:
        rowidx_np, c_np = _static_indices_np()
        return jnp.asarray(rowidx_np), jnp.asarray(c_np)
    except Exception:
        return _index_arrays()


def _body(x2, rowidx, cidx, out, idx0, idx1, in0, in1, out0, out1, cv,
          gsem0, gsem1, osem0, osem1):
    wid = lax.axis_index("s") * 2 + lax.axis_index("c")
    base = wid * _ROWS_PER_W

    pltpu.sync_copy(cidx, cv)
    cvecs = [cv[pl.ds(_LANES * j, _LANES)] for j in range(_M // _LANES)]

    def col_gather(src, dst):
        def row_body(r, carry):
            rv = jnp.full((_LANES,), r, jnp.int32)
            vals = [plsc.load_gather(src, [rv, cvecs[j]])
                    for j in range(_M // _LANES)]
            for j, v in enumerate(vals):
                dst[r, pl.ds(_LANES * j, _LANES)] = v
            return carry
        lax.fori_loop(0, _K, row_body, 0)

    def pair_body(g, carry):
        o0 = base + (2 * g) * _K
        o1 = base + (2 * g + 1) * _K
        pltpu.sync_copy(rowidx.at[pl.ds(o0, _K)], idx0)
        d0 = pltpu.async_copy(x2.at[idx0], in0, gsem0)
        pltpu.sync_copy(rowidx.at[pl.ds(o1, _K)], idx1)
        d1 = pltpu.async_copy(x2.at[idx1], in1, gsem1)
        d0.wait()
        col_gather(in0, out0)
        w0 = pltpu.async_copy(out0, out.at[pl.ds(o0, _K)], osem0)
        d1.wait()
        col_gather(in1, out1)
        w1 = pltpu.async_copy(out1, out.at[pl.ds(o1, _K)], osem1)
        w0.wait()
        w1.wait()
        return carry

    lax.fori_loop(0, _CHUNKS // 2, pair_body, 0)


def kernel(x):
    rowidx, cidx = _static_indices()
    x2 = x.reshape(_B * _C * _H, _W)

    run = pl.kernel(
        _body,
        out_type=jax.ShapeDtypeStruct((_B * _C * _M, _M), jnp.float32),
        mesh=plsc.VectorSubcoreMesh(core_axis_name="c", subcore_axis_name="s"),
        compiler_params=pltpu.CompilerParams(
            use_tc_tiling_on_sc=True, needs_layout_passes=False),
        scratch_types=[
            pltpu.VMEM((_K,), jnp.int32),
            pltpu.VMEM((_K,), jnp.int32),
            pltpu.VMEM((_K, _W), jnp.float32),
            pltpu.VMEM((_K, _W), jnp.float32),
            pltpu.VMEM((_K, _M), jnp.float32),
            pltpu.VMEM((_K, _M), jnp.float32),
            pltpu.VMEM((_M,), jnp.int32),
            pltpu.SemaphoreType.DMA,
            pltpu.SemaphoreType.DMA,
            pltpu.SemaphoreType.DMA,
            pltpu.SemaphoreType.DMA,
        ],
    )
    out2 = run(x2, rowidx, cidx)
    return out2.reshape(_B, _C, _M, _M)


# speedup vs baseline: 1.3898x; 1.3161x over previous
import functools
import math

import jax
import jax.numpy as jnp
import numpy as np
from jax import lax
from jax.experimental import pallas as pl
from jax.experimental.pallas import tpu as pltpu
from jax.experimental.pallas import tpu_sc as plsc

_GRID = 4
_STRIDE = 2

_B, _C, _H, _W = 8, 96, 384, 384
_M = 192
_NW = 32
_K = 64
_NB = 3
_ROWS_PER_W = (_B * _C * _M) // _NW
_CHUNKS = _ROWS_PER_W // _K
_TRIOS = _CHUNKS // _NB
_PLANES_PER_W = _ROWS_PER_W // _M
_LANES = 16
_NJ = _M // _LANES


def _sampled_axis_idx(key, size, grid, m):
    nblocks = math.ceil(size / grid)
    keys = jax.random.split(key, nblocks)
    perms = jax.vmap(lambda k: jax.random.permutation(k, grid)[:m])(keys)
    idx = (perms + jnp.arange(nblocks)[:, None] * grid).reshape(-1)
    return jnp.clip(jnp.sort(idx), 0, size - 1)


def _index_arrays():
    m = _GRID // _STRIDE
    kr, kc = jax.random.split(jax.random.key(42))
    r = _sampled_axis_idx(kr, _H, _GRID, m).astype(jnp.int32)
    c = _sampled_axis_idx(kc, _W, _GRID, m).astype(jnp.int32)
    return r, c


@functools.lru_cache(maxsize=None)
def _static_indices_np():
    with jax.ensure_compile_time_eval():
        r, c = _index_arrays()
        return np.asarray(r), np.asarray(c)


def _static_indices():
    try:
        r_np, c_np = _static_indices_np()
        return jnp.asarray(r_np), jnp.asarray(c_np)
    except Exception:
        return _index_arrays()


def _body(x2, ridx, cidx, out, rv_, cv_, idxs, ins, outs, gsems, osems):
    wid = lax.axis_index("s") * 2 + lax.axis_index("c")
    base = wid * _ROWS_PER_W
    plane0 = wid * _PLANES_PER_W

    pltpu.sync_copy(ridx, rv_)
    pltpu.sync_copy(cidx, cv_)
    cvecs = [cv_[pl.ds(_LANES * j, _LANES)] for j in range(_NJ)]

    def fill_idx(b, plane):
        pbase = jnp.full((_LANES,), plane * _H, jnp.int32)
        for t in range(_K // _LANES):
            off = (_K * b + _LANES * t) % _M
            idxs[b][pl.ds(_LANES * t, _LANES)] = (
                rv_[pl.ds(off, _LANES)] + pbase)

    def issue_gather(b):
        pltpu.async_copy(x2.at[idxs[b]], ins[b], gsems[b])

    def wait_gather(b):
        pltpu.make_async_copy(x2.at[idxs[b]], ins[b], gsems[b]).wait()

    def issue_out(b, o):
        pltpu.async_copy(outs[b], out.at[pl.ds(o, _K)], osems[b])

    def wait_out(b, o):
        pltpu.make_async_copy(outs[b], out.at[pl.ds(o, _K)], osems[b]).wait()

    def compute(b):
        src, dst = ins[b], outs[b]

        def row_body(r, carry):
            rv = jnp.full((_LANES,), r, jnp.int32)
            vals = [plsc.load_gather(src, [rv, cvecs[j]]) for j in range(_NJ)]
            for j, v in enumerate(vals):
                dst[r, pl.ds(_LANES * j, _LANES)] = v
            return carry
        lax.fori_loop(0, _K, row_body, 0)

    for b in range(_NB):
        fill_idx(b, plane0)
        issue_gather(b)

    for b in range(_NB):
        wait_gather(b)
        compute(b)
        issue_out(b, base + b * _K)
        fill_idx(b, plane0 + 1)
        issue_gather(b)

    def trio_body(g, carry):
        for b in range(_NB):
            c = _NB * g + b
            wait_gather(b)
            wait_out(b, base + (c - _NB) * _K)
            compute(b)
            issue_out(b, base + c * _K)
            fill_idx(b, plane0 + g + 1)
            issue_gather(b)
        return carry

    lax.fori_loop(1, _TRIOS - 1, trio_body, 0)

    for b in range(_NB):
        c = _NB * (_TRIOS - 1) + b
        wait_gather(b)
        wait_out(b, base + (c - _NB) * _K)
        compute(b)
        issue_out(b, base + c * _K)
    for b in range(_NB):
        wait_out(b, base + (_NB * (_TRIOS - 1) + b) * _K)


def kernel(x):
    ridx, cidx = _static_indices()
    x2 = x.reshape(_B * _C * _H, _W)

    run = pl.kernel(
        _body,
        out_type=jax.ShapeDtypeStruct((_B * _C * _M, _M), jnp.float32),
        mesh=plsc.VectorSubcoreMesh(core_axis_name="c", subcore_axis_name="s"),
        compiler_params=pltpu.CompilerParams(
            use_tc_tiling_on_sc=True, needs_layout_passes=False),
        scratch_types=[
            pltpu.VMEM((_M,), jnp.int32),
            pltpu.VMEM((_M,), jnp.int32),
            [pltpu.VMEM((_K,), jnp.int32) for _ in range(_NB)],
            [pltpu.VMEM((_K, _W), jnp.float32) for _ in range(_NB)],
            [pltpu.VMEM((_K, _M), jnp.float32) for _ in range(_NB)],
            [pltpu.SemaphoreType.DMA for _ in range(_NB)],
            [pltpu.SemaphoreType.DMA for _ in range(_NB)],
        ],
    )
    out2 = run(x2, ridx, cidx)
    return out2.reshape(_B, _C, _M, _M)
